# Initial kernel scaffold; baseline (speedup 1.0000x reference)
import functools

import jax
import jax.numpy as jnp
from jax import lax
from jax.experimental import pallas as pl
from jax.experimental.pallas import tpu as pltpu
from jax.experimental.pallas import tpu_sc as plsc

_V = 1000
_N = 51200
_NC, _NS = 2, 16
_NW = _NC * _NS
_BPW = _N // _NW
_CH = 32
_NCH = _BPW // _CH
_LSE_PAD = 1024


def _row_lse_body(tab_ref, out_ref):
    x = tab_ref[...]
    m = jnp.max(x, axis=1)
    s = jnp.sum(jnp.exp(x - m[:, None]), axis=1)
    out_ref[...] = m + jnp.log(s)


_row_lse = pl.pallas_call(
    _row_lse_body,
    out_shape=jax.ShapeDtypeStruct((_V,), jnp.float32),
)


def _finalize_body(p_ref, o_ref):
    o_ref[...] = jnp.sum(p_ref[...], axis=(0, 1), keepdims=True) * (1.0 / _N)


_finalize = pl.pallas_call(
    _finalize_body,
    out_shape=jax.ShapeDtypeStruct((1, 1), jnp.float32),
)


def _sc_body(table_hbm, idx_hbm, tgt_hbm, lse_hbm, out_hbm, part_hbm,
             idx_v, tgt_v, lse_v, rows0, rows1, accbuf, sem0, sem1):
    wid = lax.axis_index("s") * _NC + lax.axis_index("c")
    base = wid * _BPW

    pltpu.sync_copy(idx_hbm.at[pl.ds(base, _BPW)], idx_v)
    pltpu.sync_copy(tgt_hbm.at[pl.ds(base, _BPW)], tgt_v)
    pltpu.sync_copy(lse_hbm, lse_v)

    def gather_start(c, buf, sem):
        pltpu.make_async_copy(
            table_hbm.at[idx_v.at[pl.ds(c * _CH, _CH)]], buf, sem).start()

    def gather_wait(c, buf, sem):
        pltpu.make_async_copy(
            table_hbm.at[idx_v.at[pl.ds(c * _CH, _CH)]], buf, sem).wait()

    def consume(c, buf, acc):
        for g in range(_CH // 16):
            off = c * _CH + g * 16
            rid = lax.iota(jnp.int32, 16) + g * 16
            cols = tgt_v[pl.ds(off, 16)]
            picked = plsc.load_gather(buf, [rid, cols])
            li = idx_v[pl.ds(off, 16)]
            lses = plsc.load_gather(lse_v, [li])
            acc = acc + (lses - picked)
        pltpu.sync_copy(buf, out_hbm.at[pl.ds(base + c * _CH, _CH)])
        return acc

    gather_start(0, rows0, sem0)

    def body(i, acc):
        g = i * 2
        gather_start(g + 1, rows1, sem1)
        gather_wait(g, rows0, sem0)
        acc = consume(g, rows0, acc)

        @pl.when(g + 2 < _NCH)
        def _():
            gather_start(g + 2, rows0, sem0)

        gather_wait(g + 1, rows1, sem1)
        return consume(g + 1, rows1, acc)

    acc = lax.fori_loop(0, _NCH // 2, body, jnp.zeros((16,), jnp.float32))
    accbuf[...] = acc
    pltpu.sync_copy(accbuf, part_hbm.at[wid])


_sc_gather = functools.partial(
    pl.kernel,
    out_type=(
        jax.ShapeDtypeStruct((_N, _V), jnp.float32),
        jax.ShapeDtypeStruct((_NW, 16), jnp.float32),
    ),
    mesh=plsc.VectorSubcoreMesh(core_axis_name="c", subcore_axis_name="s"),
    scratch_types=[
        pltpu.VMEM((_BPW,), jnp.int32),
        pltpu.VMEM((_BPW,), jnp.int32),
        pltpu.VMEM((_LSE_PAD,), jnp.float32),
        pltpu.VMEM((_CH, _V), jnp.float32),
        pltpu.VMEM((_CH, _V), jnp.float32),
        pltpu.VMEM((16,), jnp.float32),
        pltpu.SemaphoreType.DMA,
        pltpu.SemaphoreType.DMA,
    ],
)(_sc_body)


def kernel(idx, targets, table):
    idx_flat = idx.reshape(-1)
    tgt_flat = targets.reshape(-1)
    lse = _row_lse(table)
    lse_pad = jnp.pad(lse, (0, _LSE_PAD - _V))
    logits2, partials = _sc_gather(table, idx_flat, tgt_flat, lse_pad)
    losses = _finalize(partials)[0, 0]
    return (logits2, losses)


# speedup vs baseline: 1.4845x; 1.4845x over previous
import functools

import jax
import jax.numpy as jnp
from jax import lax
from jax.experimental import pallas as pl
from jax.experimental.pallas import tpu as pltpu
from jax.experimental.pallas import tpu_sc as plsc

_V = 1000
_N = 51200
_NC, _NS = 2, 16
_NW = _NC * _NS
_BPW = _N // _NW
_CH = 32
_NCH = _BPW // _CH
_LSE_PAD = 1024


def _row_lse_body(tab_ref, out_ref):
    x = tab_ref[...]
    m = jnp.max(x, axis=1)
    s = jnp.sum(jnp.exp(x - m[:, None]), axis=1)
    out_ref[...] = m + jnp.log(s)


_row_lse = pl.pallas_call(
    _row_lse_body,
    out_shape=jax.ShapeDtypeStruct((_V,), jnp.float32),
)


def _finalize_body(p_ref, o_ref):
    o_ref[...] = jnp.sum(p_ref[...], axis=(0, 1), keepdims=True) * (1.0 / _N)


_finalize = pl.pallas_call(
    _finalize_body,
    out_shape=jax.ShapeDtypeStruct((1, 1), jnp.float32),
)


def _sc_body(table_hbm, idx_hbm, tgt_hbm, lse_hbm, out_hbm, part_hbm,
             idx_v, tgt_v, lse_v, rows0, rows1, accbuf, sem0, sem1):
    wid = lax.axis_index("s") * _NC + lax.axis_index("c")
    base = wid * _BPW

    pltpu.sync_copy(idx_hbm.at[pl.ds(base, _BPW)], idx_v)
    pltpu.sync_copy(tgt_hbm.at[pl.ds(base, _BPW)], tgt_v)
    pltpu.sync_copy(lse_hbm, lse_v)

    def gather_start(c, buf, sem):
        pltpu.make_async_copy(
            table_hbm.at[idx_v.at[pl.ds(c * _CH, _CH)]], buf, sem).start()

    def gather_wait(c, buf, sem):
        pltpu.make_async_copy(
            table_hbm.at[idx_v.at[pl.ds(c * _CH, _CH)]], buf, sem).wait()

    def consume(c, buf, acc):
        for g in range(_CH // 16):
            off = c * _CH + g * 16
            rid = lax.iota(jnp.int32, 16) + g * 16
            cols = tgt_v[pl.ds(off, 16)]
            picked = plsc.load_gather(buf, [rid, cols])
            li = idx_v[pl.ds(off, 16)]
            lses = plsc.load_gather(lse_v, [li])
            acc = acc + (lses - picked)
        pltpu.sync_copy(buf, out_hbm.at[pl.ds(base + c * _CH, _CH)])
        return acc

    gather_start(0, rows0, sem0)

    def body(i, acc):
        g = i * 2
        gather_start(g + 1, rows1, sem1)
        gather_wait(g, rows0, sem0)
        acc = consume(g, rows0, acc)

        @pl.when(g + 2 < _NCH)
        def _():
            gather_start(g + 2, rows0, sem0)

        gather_wait(g + 1, rows1, sem1)
        return consume(g + 1, rows1, acc)

    acc = lax.fori_loop(0, _NCH // 2, body, jnp.zeros((16,), jnp.float32))
    accbuf[...] = acc
    pltpu.sync_copy(accbuf, part_hbm.at[wid])


_sc_gather = functools.partial(
    pl.kernel,
    out_type=(
        jax.ShapeDtypeStruct((_N, _V), jnp.float32),
        jax.ShapeDtypeStruct((_NW, 16), jnp.float32),
    ),
    mesh=plsc.VectorSubcoreMesh(core_axis_name="c", subcore_axis_name="s"),
    compiler_params=pltpu.CompilerParams(
        use_tc_tiling_on_sc=False, needs_layout_passes=False),
    scratch_types=[
        pltpu.VMEM((_BPW,), jnp.int32),
        pltpu.VMEM((_BPW,), jnp.int32),
        pltpu.VMEM((_LSE_PAD,), jnp.float32),
        pltpu.VMEM((_CH, _V), jnp.float32),
        pltpu.VMEM((_CH, _V), jnp.float32),
        pltpu.VMEM((16,), jnp.float32),
        pltpu.SemaphoreType.DMA,
        pltpu.SemaphoreType.DMA,
    ],
)(_sc_body)


def kernel(idx, targets, table):
    idx_flat = idx.reshape(-1)
    tgt_flat = targets.reshape(-1)
    lse = _row_lse(table)
    lse_pad = jnp.pad(lse, (0, _LSE_PAD - _V))
    logits2, partials = _sc_gather(table, idx_flat, tgt_flat, lse_pad)
    losses = _finalize(partials)[0, 0]
    return (logits2, losses)


# speedup vs baseline: 2.2461x; 1.5130x over previous
import functools

import jax
import jax.numpy as jnp
from jax import lax
from jax.experimental import pallas as pl
from jax.experimental.pallas import tpu as pltpu
from jax.experimental.pallas import tpu_sc as plsc

_V = 1000
_VP = 1024
_N = 51200
_NC, _NS = 2, 16
_NW = _NC * _NS
_BPW = _N // _NW
_CH = 32
_NCH = _BPW // _CH

_AUX_CHUNKS = [(k * 128, 128) for k in range(_BPW // 128)]
if _BPW % 128:
    _AUX_CHUNKS.append((_BPW - _BPW % 128, _BPW % 128))


def _row_lse_body(tab_ref, out_ref):
    x = tab_ref[...]
    m = jnp.max(x, axis=1)
    s = jnp.sum(jnp.exp(x - m[:, None]), axis=1)
    out_ref[...] = m + jnp.log(s)


_row_lse = pl.pallas_call(
    _row_lse_body,
    out_shape=jax.ShapeDtypeStruct((_V,), jnp.float32),
)


def _finalize_body(p_ref, o_ref):
    o_ref[...] = jnp.sum(p_ref[...], axis=(0, 1), keepdims=True) * (1.0 / _N)


_finalize = pl.pallas_call(
    _finalize_body,
    out_shape=jax.ShapeDtypeStruct((1, 1), jnp.float32),
)


def _worker_id():
    return lax.axis_index("s") * _NC + lax.axis_index("c")


def _sc_gather_body(table_hbm, idx_hbm, out_hbm, idx_v, rows0, rows1,
                    sem0, sem1):
    base = _worker_id() * _BPW
    pltpu.sync_copy(idx_hbm.at[pl.ds(base, _BPW)], idx_v)

    def gather_start(c, buf, sem):
        pltpu.make_async_copy(
            table_hbm.at[idx_v.at[pl.ds(c * _CH, _CH)]], buf, sem).start()

    def gather_wait(c, buf, sem):
        pltpu.make_async_copy(
            table_hbm.at[idx_v.at[pl.ds(c * _CH, _CH)]], buf, sem).wait()

    def consume(c, buf):
        pltpu.sync_copy(buf, out_hbm.at[pl.ds(base + c * _CH, _CH)])

    gather_start(0, rows0, sem0)

    def body(i, carry):
        g = i * 2
        gather_start(g + 1, rows1, sem1)
        gather_wait(g, rows0, sem0)
        consume(g, rows0)

        @pl.when(g + 2 < _NCH)
        def _():
            gather_start(g + 2, rows0, sem0)

        gather_wait(g + 1, rows1, sem1)
        consume(g + 1, rows1)
        return carry

    lax.fori_loop(0, _NCH // 2, body, jnp.int32(0))


_sc_gather = functools.partial(
    pl.kernel,
    out_type=jax.ShapeDtypeStruct((_N, _VP), jnp.float32),
    mesh=plsc.VectorSubcoreMesh(core_axis_name="c", subcore_axis_name="s"),
    scratch_types=[
        pltpu.VMEM((_BPW,), jnp.int32),
        pltpu.VMEM((_CH, _VP), jnp.float32),
        pltpu.VMEM((_CH, _VP), jnp.float32),
        pltpu.SemaphoreType.DMA,
        pltpu.SemaphoreType.DMA,
    ],
)(_sc_gather_body)


def _sc_loss_body(tabflat_hbm, idx_hbm, tgt_hbm, lse_hbm, part_hbm,
                  idx_v, fi_v, picked_v, lseg_v, accbuf, sem):
    wid = _worker_id()
    base = wid * _BPW
    pltpu.sync_copy(idx_hbm.at[pl.ds(base, _BPW)], idx_v)
    pltpu.sync_copy(tgt_hbm.at[pl.ds(base, _BPW)], fi_v)

    def fi_body(i, c):
        ds = pl.ds(i * 16, 16)
        fi_v[ds] = idx_v[ds] * _V + fi_v[ds]
        return c

    lax.fori_loop(0, _BPW // 16, fi_body, jnp.int32(0))

    aux = []
    for off, ln in _AUX_CHUNKS:
        aux.append(pltpu.make_async_copy(
            tabflat_hbm.at[fi_v.at[pl.ds(off, ln)]],
            picked_v.at[pl.ds(off, ln)], sem))
        aux.append(pltpu.make_async_copy(
            lse_hbm.at[idx_v.at[pl.ds(off, ln)]],
            lseg_v.at[pl.ds(off, ln)], sem))
    for a in aux:
        a.start()
    for a in aux:
        a.wait()

    def acc_body(i, acc):
        ds = pl.ds(i * 16, 16)
        return acc + (lseg_v[ds] - picked_v[ds])

    acc = lax.fori_loop(0, _BPW // 16, acc_body, jnp.zeros((16,), jnp.float32))
    accbuf[...] = acc
    pltpu.sync_copy(accbuf, part_hbm.at[wid])


_sc_loss = functools.partial(
    pl.kernel,
    out_type=jax.ShapeDtypeStruct((_NW, 16), jnp.float32),
    mesh=plsc.VectorSubcoreMesh(core_axis_name="c", subcore_axis_name="s"),
    compiler_params=pltpu.CompilerParams(
        use_tc_tiling_on_sc=False, needs_layout_passes=False),
    scratch_types=[
        pltpu.VMEM((_BPW,), jnp.int32),
        pltpu.VMEM((_BPW,), jnp.int32),
        pltpu.VMEM((_BPW,), jnp.float32),
        pltpu.VMEM((_BPW,), jnp.float32),
        pltpu.VMEM((16,), jnp.float32),
        pltpu.SemaphoreType.DMA,
    ],
)(_sc_loss_body)


def kernel(idx, targets, table):
    idx_flat = idx.reshape(-1)
    tgt_flat = targets.reshape(-1)
    table_pad = jnp.pad(table, ((0, 0), (0, _VP - _V)))
    lse = _row_lse(table)
    lse_pad = jnp.pad(lse, (0, 24))
    out_pad = _sc_gather(table_pad, idx_flat)
    partials = _sc_loss(table.reshape(-1), idx_flat, tgt_flat, lse_pad)
    logits2 = out_pad[:, :_V]
    losses = _finalize(partials)[0, 0]
    return (logits2, losses)
